# Initial kernel scaffold; baseline (speedup 1.0000x reference)
import jax
import jax.numpy as jnp
from jax.experimental import pallas as pl


def kernel(query_embedding, episode_embeddings, temporal_weights, top_k):
    raise NotImplementedError("write your pallas kernel here")


# speedup vs baseline: 1.8042x; 1.8042x over previous
import jax
import jax.numpy as jnp
from jax import lax
from jax.experimental import pallas as pl
from jax.experimental.pallas import tpu as pltpu, tpu_sc as plsc

K = 1_000_000
D = 64
NW = 32
CH = 512
NFULL = K // CH
TAIL = K - NFULL * CH
TAIL_BASE = NFULL * CH
NCHUNK = NFULL + 1
ITERS = (NCHUNK + NW - 1) // NW
EPS = 1e-8
NEG = -1e30

_mesh = plsc.VectorSubcoreMesh(
    core_axis_name="c", subcore_axis_name="s", num_cores=2, num_subcores=16
)
_params = pltpu.CompilerParams(needs_layout_passes=False)


def _rsqrt(x):
    bits = plsc.bitcast(x, jnp.int32)
    y = plsc.bitcast(jnp.int32(0x5F3759DF) - (bits >> 1), jnp.float32)
    for _ in range(3):
        y = y * (1.5 - 0.5 * x * y * y)
    return y


def _score_body(q_hbm, ep_hbm, tw_hbm, outv_hbm, outi_hbm,
                qbuf, rbuf, wbuf, ovb, oib):
    wid = lax.axis_index("s") * 2 + lax.axis_index("c")
    iota = lax.iota(jnp.int32, 16)

    pltpu.sync_copy(q_hbm, qbuf)
    q0 = qbuf[pl.ds(0, 16)]
    q1 = qbuf[pl.ds(16, 16)]
    q2 = qbuf[pl.ds(32, 16)]
    q3 = qbuf[pl.ds(48, 16)]
    qsq = jnp.sum(q0 * q0 + q1 * q1 + q2 * q2 + q3 * q3)

    def chunk_body(c, carry):
        rv, ri = carry
        chunk = c * NW + wid
        base = pl.multiple_of(chunk * CH, CH)

        @pl.when(chunk < NFULL)
        def _():
            pltpu.sync_copy(ep_hbm.at[pl.ds(base, CH)], rbuf)
            pltpu.sync_copy(tw_hbm.at[pl.ds(base, CH)], wbuf)

        @pl.when(chunk == NFULL)
        def _():
            pltpu.sync_copy(ep_hbm.at[pl.ds(TAIL_BASE, TAIL)],
                            rbuf.at[pl.ds(0, TAIL)])
            pltpu.sync_copy(tw_hbm.at[pl.ds(TAIL_BASE, TAIL)],
                            wbuf.at[pl.ds(0, TAIL)])

        def merge_body(g, carry):
            rv, ri = carry
            off = pl.multiple_of(g * 16, 16)

            def row_body(r, dn):
                dvec, nvec = dn
                i = off + r
                r0 = rbuf[i, pl.ds(0, 16)]
                r1 = rbuf[i, pl.ds(16, 16)]
                r2 = rbuf[i, pl.ds(32, 16)]
                r3 = rbuf[i, pl.ds(48, 16)]
                dd = r0 * q0 + r1 * q1 + r2 * q2 + r3 * q3
                nn = r0 * r0 + r1 * r1 + r2 * r2 + r3 * r3
                lane = iota == r
                dvec = jnp.where(lane, jnp.sum(dd), dvec)
                nvec = jnp.where(lane, jnp.sum(nn), nvec)
                return (dvec, nvec)

            zero = jnp.zeros((16,), jnp.float32)
            d, n = lax.fori_loop(0, 16, row_body, (zero, zero))
            w = wbuf[pl.ds(off, 16)]
            gidx = base + off + iota
            y = jnp.minimum(_rsqrt(n * qsq), 1.0 / EPS)
            s = jnp.where(gidx < K, d * w * y, NEG)
            sv, si = plsc.sort_key_val(s, gidx)
            bv = lax.rev(sv, (0,))
            bi = lax.rev(si, (0,))
            keep = rv >= bv
            mv = jnp.where(keep, rv, bv)
            mi = jnp.where(keep, ri, bi)
            return tuple(plsc.sort_key_val(mv, mi))

        return lax.fori_loop(0, CH // 16, merge_body, (rv, ri))

    rv0 = jnp.full((16,), NEG, jnp.float32)
    ri0 = jnp.zeros((16,), jnp.int32)
    rv, ri = lax.fori_loop(0, ITERS, chunk_body, (rv0, ri0))

    ovb[...] = rv
    oib[...] = ri
    pltpu.sync_copy(ovb, outv_hbm.at[wid])
    pltpu.sync_copy(oib, outi_hbm.at[wid])


_score = pl.kernel(
    _score_body,
    out_type=(
        jax.ShapeDtypeStruct((NW, 16), jnp.float32),
        jax.ShapeDtypeStruct((NW, 16), jnp.int32),
    ),
    mesh=_mesh,
    scratch_types=[
        pltpu.VMEM((D,), jnp.float32),
        pltpu.VMEM((CH, D), jnp.float32),
        pltpu.VMEM((CH,), jnp.float32),
        pltpu.VMEM((16,), jnp.float32),
        pltpu.VMEM((16,), jnp.int32),
    ],
    compiler_params=_params,
)


def _merge_body(cv_hbm, ci_hbm, outv_hbm, outi_hbm, cvb, cib, ovb, oib):
    wid = lax.axis_index("s") * 2 + lax.axis_index("c")

    @pl.when(wid == 0)
    def _():
        pltpu.sync_copy(cv_hbm, cvb)
        pltpu.sync_copy(ci_hbm, cib)
        rv = cvb[0, pl.ds(0, 16)]
        ri = cib[0, pl.ds(0, 16)]
        for j in range(1, NW):
            bv = lax.rev(cvb[j, pl.ds(0, 16)], (0,))
            bi = lax.rev(cib[j, pl.ds(0, 16)], (0,))
            keep = rv >= bv
            mv = jnp.where(keep, rv, bv)
            mi = jnp.where(keep, ri, bi)
            rv, ri = plsc.sort_key_val(mv, mi)
        fv, fi = plsc.sort_key_val(rv, ri, descending=True)
        ovb[...] = fv
        oib[...] = fi
        pltpu.sync_copy(ovb, outv_hbm)
        pltpu.sync_copy(oib, outi_hbm)


_merge = pl.kernel(
    _merge_body,
    out_type=(
        jax.ShapeDtypeStruct((16,), jnp.float32),
        jax.ShapeDtypeStruct((16,), jnp.int32),
    ),
    mesh=_mesh,
    scratch_types=[
        pltpu.VMEM((NW, 16), jnp.float32),
        pltpu.VMEM((NW, 16), jnp.int32),
        pltpu.VMEM((16,), jnp.float32),
        pltpu.VMEM((16,), jnp.int32),
    ],
    compiler_params=_params,
)


def kernel(query_embedding, episode_embeddings, temporal_weights, top_k):
    del top_k
    cv, ci = _score(query_embedding, episode_embeddings, temporal_weights)
    fv, fi = _merge(cv, ci)
    return fv[:5], fi[:5]


# speedup vs baseline: 2.0601x; 1.1419x over previous
import jax
import jax.numpy as jnp
from jax import lax
from jax.experimental import pallas as pl
from jax.experimental.pallas import tpu as pltpu, tpu_sc as plsc

K = 1_000_000
D = 64
NW = 32
CH = 512
NFULL = K // CH
TAIL = K - NFULL * CH
TAIL_BASE = NFULL * CH
NCHUNK = NFULL + 1
ITERS = (NCHUNK + NW - 1) // NW
EPS = 1e-8
NEG = -1e30

_mesh = plsc.VectorSubcoreMesh(
    core_axis_name="c", subcore_axis_name="s", num_cores=2, num_subcores=16
)
_params = pltpu.CompilerParams(needs_layout_passes=False)


def _rsqrt(x):
    bits = plsc.bitcast(x, jnp.int32)
    y = plsc.bitcast(jnp.int32(0x5F3759DF) - (bits >> 1), jnp.float32)
    for _ in range(3):
        y = y * (1.5 - 0.5 * x * y * y)
    return y


def _score_body(q_hbm, ep_hbm, tw_hbm, outv_hbm, outi_hbm,
                qbuf, rbuf, wbuf, ovb, oib):
    wid = lax.axis_index("s") * 2 + lax.axis_index("c")
    iota = lax.iota(jnp.int32, 16)

    pltpu.sync_copy(q_hbm, qbuf)
    q0 = qbuf[pl.ds(0, 16)]
    q1 = qbuf[pl.ds(16, 16)]
    q2 = qbuf[pl.ds(32, 16)]
    q3 = qbuf[pl.ds(48, 16)]
    qsq = jnp.sum(q0 * q0 + q1 * q1 + q2 * q2 + q3 * q3)

    def chunk_body(c, carry):
        rv, ri = carry
        chunk = c * NW + wid
        base = pl.multiple_of(chunk * CH, CH)

        @pl.when(chunk < NFULL)
        def _():
            pltpu.sync_copy(ep_hbm.at[pl.ds(base, CH)], rbuf)
            pltpu.sync_copy(tw_hbm.at[pl.ds(base, CH)], wbuf)

        @pl.when(chunk == NFULL)
        def _():
            pltpu.sync_copy(ep_hbm.at[pl.ds(TAIL_BASE, TAIL)],
                            rbuf.at[pl.ds(0, TAIL)])
            pltpu.sync_copy(tw_hbm.at[pl.ds(TAIL_BASE, TAIL)],
                            wbuf.at[pl.ds(0, TAIL)])

        def merge_body(g, carry):
            rv, ri = carry
            off = pl.multiple_of(g * 16, 16)

            zero = jnp.zeros((16,), jnp.float32)
            d, n = zero, zero
            for r in range(16):
                i = off + r
                r0 = rbuf[i, pl.ds(0, 16)]
                r1 = rbuf[i, pl.ds(16, 16)]
                r2 = rbuf[i, pl.ds(32, 16)]
                r3 = rbuf[i, pl.ds(48, 16)]
                dd = r0 * q0 + r1 * q1 + r2 * q2 + r3 * q3
                nn = r0 * r0 + r1 * r1 + r2 * r2 + r3 * r3
                lane = iota == r
                d = jnp.where(lane, jnp.sum(dd), d)
                n = jnp.where(lane, jnp.sum(nn), n)
            w = wbuf[pl.ds(off, 16)]
            gidx = base + off + iota
            y = jnp.minimum(_rsqrt(n * qsq), 1.0 / EPS)
            s = jnp.where(gidx < K, d * w * y, NEG)
            sv, si = plsc.sort_key_val(s, gidx)
            bv = lax.rev(sv, (0,))
            bi = lax.rev(si, (0,))
            keep = rv >= bv
            mv = jnp.where(keep, rv, bv)
            mi = jnp.where(keep, ri, bi)
            return tuple(plsc.sort_key_val(mv, mi))

        return lax.fori_loop(0, CH // 16, merge_body, (rv, ri))

    rv0 = jnp.full((16,), NEG, jnp.float32)
    ri0 = jnp.zeros((16,), jnp.int32)
    rv, ri = lax.fori_loop(0, ITERS, chunk_body, (rv0, ri0))

    ovb[...] = rv
    oib[...] = ri
    pltpu.sync_copy(ovb, outv_hbm.at[wid])
    pltpu.sync_copy(oib, outi_hbm.at[wid])


_score = pl.kernel(
    _score_body,
    out_type=(
        jax.ShapeDtypeStruct((NW, 16), jnp.float32),
        jax.ShapeDtypeStruct((NW, 16), jnp.int32),
    ),
    mesh=_mesh,
    scratch_types=[
        pltpu.VMEM((D,), jnp.float32),
        pltpu.VMEM((CH, D), jnp.float32),
        pltpu.VMEM((CH,), jnp.float32),
        pltpu.VMEM((16,), jnp.float32),
        pltpu.VMEM((16,), jnp.int32),
    ],
    compiler_params=_params,
)


def _merge_body(cv_hbm, ci_hbm, outv_hbm, outi_hbm, cvb, cib, ovb, oib):
    wid = lax.axis_index("s") * 2 + lax.axis_index("c")

    @pl.when(wid == 0)
    def _():
        pltpu.sync_copy(cv_hbm, cvb)
        pltpu.sync_copy(ci_hbm, cib)
        rv = cvb[0, pl.ds(0, 16)]
        ri = cib[0, pl.ds(0, 16)]
        for j in range(1, NW):
            bv = lax.rev(cvb[j, pl.ds(0, 16)], (0,))
            bi = lax.rev(cib[j, pl.ds(0, 16)], (0,))
            keep = rv >= bv
            mv = jnp.where(keep, rv, bv)
            mi = jnp.where(keep, ri, bi)
            rv, ri = plsc.sort_key_val(mv, mi)
        fv, fi = plsc.sort_key_val(rv, ri, descending=True)
        ovb[...] = fv
        oib[...] = fi
        pltpu.sync_copy(ovb, outv_hbm)
        pltpu.sync_copy(oib, outi_hbm)


_merge = pl.kernel(
    _merge_body,
    out_type=(
        jax.ShapeDtypeStruct((16,), jnp.float32),
        jax.ShapeDtypeStruct((16,), jnp.int32),
    ),
    mesh=_mesh,
    scratch_types=[
        pltpu.VMEM((NW, 16), jnp.float32),
        pltpu.VMEM((NW, 16), jnp.int32),
        pltpu.VMEM((16,), jnp.float32),
        pltpu.VMEM((16,), jnp.int32),
    ],
    compiler_params=_params,
)


def kernel(query_embedding, episode_embeddings, temporal_weights, top_k):
    del top_k
    cv, ci = _score(query_embedding, episode_embeddings, temporal_weights)
    fv, fi = _merge(cv, ci)
    return fv[:5], fi[:5]
